# Initial kernel scaffold; baseline (speedup 1.0000x reference)
import jax
import jax.numpy as jnp
from jax.experimental import pallas as pl


def kernel(pos, residue_index, chain_index, batch_index, mask, is_target, hotspots, aa_gt, params):
    raise NotImplementedError("write your pallas kernel here")


# speedup vs baseline: 4.9958x; 4.9958x over previous
import functools
import numpy as np
import jax
import jax.numpy as jnp
from jax import lax
from jax.experimental import pallas as pl
from jax.experimental.pallas import tpu as pltpu
from jax.experimental.pallas import tpu_sc as plsc

N = 4096; A = 14; D = 256; P = 64; K = 32; AUG = 8; H = 8; DH = 32; DEPTH = 2
NA = 5 + AUG
FEAT = NA * 3 + NA * 16 + NA + 2 + 21
BIG = 1e9
GEO = 128
NK = N * K

_INTERP = False


def _ln(x):
    m = jnp.mean(x, -1, keepdims=True)
    v = jnp.var(x, -1, keepdims=True)
    return (x - m) * lax.rsqrt(v + 1e-5)


def _frames_cols(X, Y, Z):
    nx, ny, nz = X[:, 0:1], Y[:, 0:1], Z[:, 0:1]
    cax, cay, caz = X[:, 1:2], Y[:, 1:2], Z[:, 1:2]
    cx, cy, cz = X[:, 2:3], Y[:, 2:3], Z[:, 2:3]
    e1x, e1y, e1z = cx - cax, cy - cay, cz - caz
    n1 = jnp.sqrt(e1x * e1x + e1y * e1y + e1z * e1z)
    e1x, e1y, e1z = e1x / (n1 + 1e-8), e1y / (n1 + 1e-8), e1z / (n1 + 1e-8)
    ux, uy, uz = nx - cax, ny - cay, nz - caz
    dot = ux * e1x + uy * e1y + uz * e1z
    ux, uy, uz = ux - dot * e1x, uy - dot * e1y, uz - dot * e1z
    n2 = jnp.sqrt(ux * ux + uy * uy + uz * uz)
    e2x, e2y, e2z = ux / (n2 + 1e-8), uy / (n2 + 1e-8), uz / (n2 + 1e-8)
    e3x = e1y * e2z - e1z * e2y
    e3y = e1z * e2x - e1x * e2z
    e3z = e1x * e2y - e1y * e2x
    r = ((e1x, e2x, e3x), (e1y, e2y, e3y), (e1z, e2z, e3z))
    t = (cax, cay, caz)
    dX, dY, dZ = X - cax, Y - cay, Z - caz
    lp = tuple(dX * r[0][k] + dY * r[1][k] + dZ * r[2][k] for k in range(3))
    return r, t, lp


def _rbf_cols(dd, dmin, dmax, nb):
    c = np.linspace(dmin, dmax, nb)
    s = (dmax - dmin) / nb
    return jnp.concatenate(
        [jnp.exp(-((dd - float(cj)) ** 2) / (2 * s * s)) for cj in c], axis=1)


def _perm_feats():
    p = []
    for k in range(3):
        for a in range(NA):
            p.append(a * 3 + k)
    for j in range(16):
        for a in range(NA):
            p.append(3 * NA + a * 16 + j)
    p.extend(range(3 * NA + 16 * NA, FEAT))
    return np.array(p)


def _perm_lp():
    return np.array([a * 3 + k for k in range(3) for a in range(NA)])


def _perm_cm(na, nc):
    return np.array([a * nc + j for j in range(nc) for a in range(na)])


_S = np.zeros((D, H), np.float32)
for _h in range(H):
    _S[_h * DH:(_h + 1) * DH, _h] = 1.0


def _prep_params(params):
    pf = _perm_feats()
    plp = _perm_lp()
    pcm = _perm_cm(NA, 16)
    out = {
        'w_augT': params['w_aug'].T,
        'wl1p': params['wl1'][pf],
        'wl2': params['wl2'],
        'blocks': []
    }
    for bp in params['blocks']:
        wcat = jnp.concatenate([
            bp['w_rp'], bp['w_d'][pcm], bp['w_dir'][plp], bp['w_rot'],
            bp['w_pv'][plp]], axis=0)
        out['blocks'].append({
            'wcat': wcat, 'pm1': bp['pm1'], 'pb1': bp['pb1'][None, :],
            'pm2': bp['pm2'], 'pb2': bp['pb2'][None, :], 'wb': bp['wb'],
            'wq': bp['wq'], 'wkv': jnp.concatenate([bp['wk'], bp['wv']], 1),
            'wo': bp['wo'], 'wp1p': bp['wp1'][plp], 'wp2': bp['wp2'],
            'wu': bp['wu'], 'wg': bp['wg'], 'wout': bp['wout']})
    return out


BLK_P = 512


def _pro_body(X, Y, Z, ist, hot, aam, chain, w_augT, wl1p, wl2,
              local_o, lp_o, geo_o):
    X, Y, Z = X[...], Y[...], Z[...]
    r, t, lp = _frames_cols(X, Y, Z)
    w_augT_ = w_augT[...]
    aug = tuple(jnp.dot(lp[d], w_augT_, preferred_element_type=jnp.float32)
                for d in range(3))
    nrm = jnp.mean(jnp.sqrt(aug[0] ** 2 + aug[1] ** 2 + aug[2] ** 2),
                   axis=-1, keepdims=True)
    auga = tuple(aug[d] / (nrm + 1e-8) for d in range(3))
    lp2 = tuple(jnp.concatenate([lp[d][:, :5], auga[d]], axis=1)
                for d in range(3))
    pos2 = tuple(t[d] + sum(r[d][k] * lp2[k] for k in range(3))
                 for d in range(3))
    r2, t2, lpl = _frames_cols(*pos2)
    dist = jnp.sqrt(lpl[0] ** 2 + lpl[1] ** 2 + lpl[2] ** 2 + 1e-8)
    aam_ = aam[...]
    oh_aa = (lax.broadcasted_iota(jnp.int32, (aam_.shape[0], 21), 1)
             == aam_).astype(jnp.float32)
    feats = jnp.concatenate(
        [lpl[0] / (dist + 1e-8), lpl[1] / (dist + 1e-8),
         lpl[2] / (dist + 1e-8), _rbf_cols(dist, 0.0, 22.0, 16),
         jnp.log(dist + 1.0), ist[...], hot[...], oh_aa], axis=1)
    h = jax.nn.gelu(jnp.dot(feats, wl1p[...],
                            preferred_element_type=jnp.float32))
    local_o[...] = _ln(jnp.dot(h, wl2[...],
                               preferred_element_type=jnp.float32))
    lp_o[...] = jnp.concatenate(lpl, axis=1)
    r2f = jnp.concatenate([r2[d][k] for d in range(3) for k in range(3)],
                          axis=1)
    zpad = jnp.zeros((X.shape[0], GEO - (3 * NA + 9 + 1)), jnp.float32)
    geo_o[...] = jnp.concatenate(list(pos2) + [r2f, chain[...], zpad], axis=1)


def _run_prologue(X, Y, Z, ist, hot, aam, chain, w_augT, wl1p, wl2):
    grid = (N // BLK_P,)
    row = lambda i: (i, 0)
    full = lambda i: (0, 0)
    return pl.pallas_call(
        _pro_body,
        grid=grid,
        in_specs=[
            pl.BlockSpec((BLK_P, A), row), pl.BlockSpec((BLK_P, A), row),
            pl.BlockSpec((BLK_P, A), row), pl.BlockSpec((BLK_P, 1), row),
            pl.BlockSpec((BLK_P, 1), row), pl.BlockSpec((BLK_P, 1), row),
            pl.BlockSpec((BLK_P, 1), row),
            pl.BlockSpec((A, AUG), full), pl.BlockSpec((FEAT, 4 * D), full),
            pl.BlockSpec((4 * D, D), full),
        ],
        out_specs=[
            pl.BlockSpec((BLK_P, D), row), pl.BlockSpec((BLK_P, 3 * NA), row),
            pl.BlockSpec((BLK_P, GEO), row),
        ],
        out_shape=[
            jax.ShapeDtypeStruct((N, D), jnp.float32),
            jax.ShapeDtypeStruct((N, 3 * NA), jnp.float32),
            jax.ShapeDtypeStruct((N, GEO), jnp.float32),
        ],
        interpret=_INTERP,
    )(X, Y, Z, ist, hot, aam, chain, w_augT, wl1p, wl2)


BLK_K = 256


def _knn_body(cxc, cyc, czc, batc, istc, cxr, cyr, czr, batr, istr,
              nb_o, nbs_o):
    cx, cy, cz = cxc[...], cyc[...], czc[...]
    sqc = cx * cx + cy * cy + cz * cz
    rx, ry, rz = cxr[...], cyr[...], czr[...]
    sqr = rx * rx + ry * ry + rz * rz
    d2 = sqc + sqr - 2.0 * (cx * rx + cy * ry + cz * rz)
    d2 = jnp.maximum(d2, 0.0)
    keep = (batc[...] == batr[...]) & ~((istc[...] == 1) & (istr[...] == 0))
    d = jnp.where(keep, d2, BIG)
    iota = lax.broadcasted_iota(jnp.int32, d.shape, 1)
    cols = []
    for _ in range(K):
        m = jnp.min(d, axis=1, keepdims=True)
        amin = jnp.min(jnp.where(d == m, iota, N), axis=1, keepdims=True)
        cols.append(jnp.where(m < BIG / 2, amin, -1))
        d = jnp.where(iota == amin, BIG, d)
    nb = jnp.concatenate(cols, axis=1)
    nb_o[...] = nb
    nbs_o[...] = jnp.maximum(nb, 0)


def _run_knn(cxc, cyc, czc, batc, istc, cxr, cyr, czr, batr, istr):
    grid = (N // BLK_K,)
    col = lambda i: (i, 0)
    full = lambda i: (0, 0)
    return pl.pallas_call(
        _knn_body,
        grid=grid,
        in_specs=[pl.BlockSpec((BLK_K, 1), col)] * 5
        + [pl.BlockSpec((1, N), full)] * 5,
        out_specs=[pl.BlockSpec((BLK_K, K), col)] * 2,
        out_shape=[jax.ShapeDtypeStruct((N, K), jnp.int32)] * 2,
        interpret=_INTERP,
    )(cxc, cyc, czc, batc, istc, cxr, cyr, czr, batr, istr)


SC_CH = 128


def _sc_gather(table, idx):
    V, Dt = table.shape
    B = idx.shape[0]
    NW = 32
    per_w = B // NW
    chunks = per_w // SC_CH
    mesh = plsc.VectorSubcoreMesh(core_axis_name="c", subcore_axis_name="s")

    @functools.partial(
        pl.kernel, mesh=mesh,
        out_type=jax.ShapeDtypeStruct((B, Dt), jnp.float32),
        scratch_types=[
            pltpu.VMEM((SC_CH,), jnp.int32),
            pltpu.VMEM((SC_CH, Dt), jnp.float32),
            pltpu.SemaphoreType.DMA,
        ],
    )
    def k(table_hbm, idx_hbm, out_hbm, idx_v, rows_v, sem):
        wid = lax.axis_index("s") * 2 + lax.axis_index("c")
        base = wid * per_w

        def body(c, carry):
            off = base + c * SC_CH
            pltpu.sync_copy(idx_hbm.at[pl.ds(off, SC_CH)], idx_v)
            pltpu.async_copy(table_hbm.at[idx_v], rows_v, sem).wait()
            pltpu.sync_copy(rows_v, out_hbm.at[pl.ds(off, SC_CH)])
            return carry

        lax.fori_loop(0, chunks, body, 0)

    return k(table, idx)


PBLK = 2048
PNODE = PBLK // K


def _pair_body(pid, geo, geo_nb, nbf, wcat, pm1, pb1, pm2, pb2, wb, bias_o):
    g = geo[...]
    gs3 = jnp.broadcast_to(g[:, None, :], (PNODE, K, GEO))
    gs = gs3.reshape(PBLK, GEO)
    gn = geo_nb[...]
    nbf_ = nbf[...]
    m_idx = jnp.maximum(nbf_, 0)
    n_idx = (pid * PNODE
             + lax.broadcasted_iota(jnp.int32, (PBLK, 1), 0) // K)
    rel = jnp.clip(m_idx - n_idx, -32, 32) + 32
    same = (gn[:, 48:49] == gs[:, 48:49]).astype(jnp.float32)
    oh = (lax.broadcasted_iota(jnp.int32, (PBLK, 65), 1)
          == rel).astype(jnp.float32) * same
    sg = [gs[:, d * NA:(d + 1) * NA] for d in range(3)]
    ng = [gn[:, d * NA:(d + 1) * NA] for d in range(3)]
    sR = [[gs[:, 3 * NA + d * 3 + e:3 * NA + d * 3 + e + 1] for e in range(3)]
          for d in range(3)]
    nR = [[gn[:, 3 * NA + d * 3 + f:3 * NA + d * 3 + f + 1] for f in range(3)]
          for d in range(3)]
    diff = [sg[d] - ng[d][:, 4:5] for d in range(3)]
    dd = jnp.sqrt(diff[0] ** 2 + diff[1] ** 2 + diff[2] ** 2 + 1e-8)
    rbf = _rbf_cols(dd, 0.0, 22.0, 16)
    dirs = jnp.concatenate(
        [sum(diff[d] * sR[d][e] for d in range(3)) / (dd + 1e-8)
         for e in range(3)], axis=1)
    rrel = jnp.concatenate(
        [sum(sR[d][e] * nR[d][f] for d in range(3))
         for e in range(3) for f in range(3)], axis=1)
    st = [sg[d][:, 1:2] for d in range(3)]
    pv = jnp.concatenate(
        [sum((ng[d] - st[d]) * sR[d][e] for d in range(3)) for e in range(3)],
        axis=1)
    f360 = jnp.concatenate([oh, rbf, dirs, rrel, pv], axis=1)
    pair = _ln(jnp.dot(f360, wcat[...], preferred_element_type=jnp.float32))
    pair = jnp.dot(
        jax.nn.gelu(jnp.dot(pair, pm1[...],
                            preferred_element_type=jnp.float32) + pb1[...]),
        pm2[...], preferred_element_type=jnp.float32) + pb2[...]
    bias = jnp.dot(pair, wb[...], preferred_element_type=jnp.float32)
    bias_o[...] = jnp.where(nbf_ >= 0, bias, -1e9)


def _run_pair(geo, geo_nb, nbf, bp):
    grid = (NK // PBLK,)
    full = lambda i: (0, 0)
    body = functools.partial(_pair_body)

    def wrapped(geo_r, geo_nb_r, nbf_r, wcat, pm1, pb1, pm2, pb2, wb, bias_o):
        body(pl.program_id(0), geo_r, geo_nb_r, nbf_r, wcat, pm1, pb1, pm2,
             pb2, wb, bias_o)

    return pl.pallas_call(
        wrapped,
        grid=grid,
        in_specs=[
            pl.BlockSpec((PNODE, GEO), lambda i: (i, 0)),
            pl.BlockSpec((PBLK, GEO), lambda i: (i, 0)),
            pl.BlockSpec((PBLK, 1), lambda i: (i, 0)),
            pl.BlockSpec((65 + 16 * NA + 3 * NA + 9 + 3 * NA, P), full),
            pl.BlockSpec((P, 2 * P), full), pl.BlockSpec((1, 2 * P), full),
            pl.BlockSpec((2 * P, P), full), pl.BlockSpec((1, P), full),
            pl.BlockSpec((P, H), full),
        ],
        out_specs=pl.BlockSpec((PBLK, H), lambda i: (i, 0)),
        out_shape=jax.ShapeDtypeStruct((NK, H), jnp.float32),
        interpret=_INTERP,
    )(geo, geo_nb, nbf, bp['wcat'], bp['pm1'], bp['pb1'], bp['pm2'],
      bp['pb2'], bp['wb'])


BLK_A = 64


def _attn_body(final, local, incr, kvn, bias, lpf, wq, wo, wp1p, wp2, wu,
               wg, wout, smat, local_o, incr_o):
    loc = local[...]
    q = jnp.dot(loc, wq[...], preferred_element_type=jnp.float32)
    kvn_ = kvn[...]
    kn3 = kvn_[:, :D].reshape(BLK_A, K, D)
    vn3 = kvn_[:, D:].reshape(BLK_A, K, D)
    prod = (q[:, None, :] * kn3).reshape(BLK_A * K, D)
    sm = smat[...]
    logits = (jnp.dot(prod, sm, preferred_element_type=jnp.float32)
              .reshape(BLK_A, K, H) / np.sqrt(DH)
              + bias[...].reshape(BLK_A, K, H))
    mx = jnp.max(logits, axis=1, keepdims=True)
    e = jnp.exp(logits - mx)
    a = e / jnp.sum(e, axis=1, keepdims=True)
    arep = jnp.dot(a.reshape(BLK_A * K, H), sm.T,
                   preferred_element_type=jnp.float32).reshape(BLK_A, K, D)
    o = jnp.sum(arep * vn3, axis=1)
    up = jnp.dot(o, wo[...], preferred_element_type=jnp.float32)
    inc = incr[...] + up
    loc = _ln(loc + up)
    l2 = loc + jnp.dot(
        jax.nn.gelu(jnp.dot(lpf[...], wp1p[...],
                            preferred_element_type=jnp.float32)),
        wp2[...], preferred_element_type=jnp.float32)
    lu = jnp.dot(l2, wu[...], preferred_element_type=jnp.float32)
    lg = jax.nn.gelu(jnp.dot(l2, wg[...], preferred_element_type=jnp.float32))
    up2 = jnp.dot(lg * lu, wout[...], preferred_element_type=jnp.float32)
    inc = inc + up2
    loc = _ln(loc + up2)
    if final:
        loc = loc + _ln(inc)
    local_o[...] = loc
    incr_o[...] = inc


def _run_attn(local, incr, kvn, bias, lpf, bp, final):
    grid = (N // BLK_A,)
    row = lambda i: (i, 0)
    full = lambda i: (0, 0)
    return pl.pallas_call(
        functools.partial(_attn_body, final),
        grid=grid,
        in_specs=[
            pl.BlockSpec((BLK_A, D), row), pl.BlockSpec((BLK_A, D), row),
            pl.BlockSpec((BLK_A * K, 2 * D), row),
            pl.BlockSpec((BLK_A * K, H), row),
            pl.BlockSpec((BLK_A, 3 * NA), row),
            pl.BlockSpec((D, D), full), pl.BlockSpec((D, D), full),
            pl.BlockSpec((3 * NA, 2 * D), full), pl.BlockSpec((2 * D, D), full),
            pl.BlockSpec((D, 2 * D), full), pl.BlockSpec((D, 2 * D), full),
            pl.BlockSpec((2 * D, D), full), pl.BlockSpec((D, H), full),
        ],
        out_specs=[pl.BlockSpec((BLK_A, D), row)] * 2,
        out_shape=[jax.ShapeDtypeStruct((N, D), jnp.float32)] * 2,
        interpret=_INTERP,
    )(local, incr, kvn, bias, lpf, bp['wq'], bp['wo'], bp['wp1p'],
      bp['wp2'], bp['wu'], bp['wg'], bp['wout'], jnp.asarray(_S))


BLK_M = 512


def _kv_body(local, wkv, kv_o):
    kv_o[...] = jnp.dot(local[...], wkv[...],
                        preferred_element_type=jnp.float32)


def _run_kv(local, wkv):
    return pl.pallas_call(
        _kv_body,
        grid=(N // BLK_M,),
        in_specs=[pl.BlockSpec((BLK_M, D), lambda i: (i, 0)),
                  pl.BlockSpec((D, 2 * D), lambda i: (0, 0))],
        out_specs=pl.BlockSpec((BLK_M, 2 * D), lambda i: (i, 0)),
        out_shape=jax.ShapeDtypeStruct((N, 2 * D), jnp.float32),
        interpret=_INTERP,
    )(local, wkv)


def kernel(pos, residue_index, chain_index, batch_index, mask, is_target,
           hotspots, aa_gt, params):
    del residue_index, mask
    pp = _prep_params(params)
    X = pos[:, :, 0]
    Y = pos[:, :, 1]
    Z = pos[:, :, 2]
    is_t = is_target.astype(jnp.int32)[:, None]
    aam = jnp.where(is_target, aa_gt, 20).astype(jnp.int32)[:, None]
    chain_f = chain_index.astype(jnp.float32)[:, None]
    local, lp_flat, geo = _run_prologue(
        X, Y, Z, is_t.astype(jnp.float32),
        hotspots.astype(jnp.float32)[:, None], aam, chain_f,
        pp['w_augT'], pp['wl1p'], pp['wl2'])
    cx = geo[:, 4:5]
    cy = geo[:, NA + 4:NA + 5]
    cz = geo[:, 2 * NA + 4:2 * NA + 5]
    bat = batch_index.astype(jnp.int32)[:, None]
    nb, nbs = _run_knn(cx, cy, cz, bat, is_t,
                       cx.reshape(1, N), cy.reshape(1, N), cz.reshape(1, N),
                       bat.reshape(1, N), is_t.reshape(1, N))
    nbs_flat = nbs.reshape(NK)
    nb_flat = nb.reshape(NK, 1)
    geo_nb = _sc_gather(geo, nbs_flat)
    incremental = local
    for i, bp in enumerate(pp['blocks']):
        bias = _run_pair(geo, geo_nb, nb_flat, bp)
        kv = _run_kv(local, bp['wkv'])
        kvn = _sc_gather(kv, nbs_flat)
        local, incremental = _run_attn(
            local, incremental, kvn, bias, lp_flat, bp,
            final=(i == DEPTH - 1))
    return local


# speedup vs baseline: 10.8416x; 2.1702x over previous
import functools
import numpy as np
import jax
import jax.numpy as jnp
from jax import lax
from jax.experimental import pallas as pl
from jax.experimental.pallas import tpu as pltpu
from jax.experimental.pallas import tpu_sc as plsc

N = 4096; A = 14; D = 256; P = 64; K = 32; AUG = 8; H = 8; DH = 32; DEPTH = 2
NA = 5 + AUG
FEAT = NA * 3 + NA * 16 + NA + 2 + 21
BIG = 1e9
GEO = 128
NK = N * K

_INTERP = False


def _ln(x):
    m = jnp.mean(x, -1, keepdims=True)
    v = jnp.var(x, -1, keepdims=True)
    return (x - m) * lax.rsqrt(v + 1e-5)


def _frames_cols(X, Y, Z):
    nx, ny, nz = X[:, 0:1], Y[:, 0:1], Z[:, 0:1]
    cax, cay, caz = X[:, 1:2], Y[:, 1:2], Z[:, 1:2]
    cx, cy, cz = X[:, 2:3], Y[:, 2:3], Z[:, 2:3]
    e1x, e1y, e1z = cx - cax, cy - cay, cz - caz
    n1 = jnp.sqrt(e1x * e1x + e1y * e1y + e1z * e1z)
    e1x, e1y, e1z = e1x / (n1 + 1e-8), e1y / (n1 + 1e-8), e1z / (n1 + 1e-8)
    ux, uy, uz = nx - cax, ny - cay, nz - caz
    dot = ux * e1x + uy * e1y + uz * e1z
    ux, uy, uz = ux - dot * e1x, uy - dot * e1y, uz - dot * e1z
    n2 = jnp.sqrt(ux * ux + uy * uy + uz * uz)
    e2x, e2y, e2z = ux / (n2 + 1e-8), uy / (n2 + 1e-8), uz / (n2 + 1e-8)
    e3x = e1y * e2z - e1z * e2y
    e3y = e1z * e2x - e1x * e2z
    e3z = e1x * e2y - e1y * e2x
    r = ((e1x, e2x, e3x), (e1y, e2y, e3y), (e1z, e2z, e3z))
    t = (cax, cay, caz)
    dX, dY, dZ = X - cax, Y - cay, Z - caz
    lp = tuple(dX * r[0][k] + dY * r[1][k] + dZ * r[2][k] for k in range(3))
    return r, t, lp


def _rbf_cols(dd, dmin, dmax, nb):
    c = np.linspace(dmin, dmax, nb)
    s = (dmax - dmin) / nb
    return jnp.concatenate(
        [jnp.exp(-((dd - float(cj)) ** 2) / (2 * s * s)) for cj in c], axis=1)


def _perm_feats():
    p = []
    for k in range(3):
        for a in range(NA):
            p.append(a * 3 + k)
    for j in range(16):
        for a in range(NA):
            p.append(3 * NA + a * 16 + j)
    p.extend(range(3 * NA + 16 * NA, FEAT))
    return np.array(p)


def _perm_lp():
    return np.array([a * 3 + k for k in range(3) for a in range(NA)])


def _perm_cm(na, nc):
    return np.array([a * nc + j for j in range(nc) for a in range(na)])


_S = np.zeros((D, H), np.float32)
for _h in range(H):
    _S[_h * DH:(_h + 1) * DH, _h] = 1.0


def _prep_params(params):
    pf = _perm_feats()
    plp = _perm_lp()
    pcm = _perm_cm(NA, 16)
    out = {
        'w_augT': params['w_aug'].T,
        'wl1p': params['wl1'][pf],
        'wl2': params['wl2'],
        'blocks': []
    }
    for bp in params['blocks']:
        wcat = jnp.concatenate([
            bp['w_rp'], bp['w_d'][pcm], bp['w_dir'][plp], bp['w_rot'],
            bp['w_pv'][plp]], axis=0)
        out['blocks'].append({
            'wcatT': wcat.T, 'pm1T': bp['pm1'].T, 'pb1T': bp['pb1'][:, None],
            'pm2T': bp['pm2'].T, 'pb2T': bp['pb2'][:, None],
            'wbT': bp['wb'].T,
            'wq': bp['wq'], 'wkv': jnp.concatenate([bp['wk'], bp['wv']], 1),
            'wo': bp['wo'], 'wp1p': bp['wp1'][plp], 'wp2': bp['wp2'],
            'wu': bp['wu'], 'wg': bp['wg'], 'wout': bp['wout']})
    return out


BLK_P = 512


def _pro_body(X, Y, Z, ist, hot, aam, chain, w_augT, wl1p, wl2,
              local_o, lp_o, geo_o):
    X, Y, Z = X[...], Y[...], Z[...]
    r, t, lp = _frames_cols(X, Y, Z)
    w_augT_ = w_augT[...]
    aug = tuple(jnp.dot(lp[d], w_augT_, preferred_element_type=jnp.float32)
                for d in range(3))
    nrm = jnp.mean(jnp.sqrt(aug[0] ** 2 + aug[1] ** 2 + aug[2] ** 2),
                   axis=-1, keepdims=True)
    auga = tuple(aug[d] / (nrm + 1e-8) for d in range(3))
    lp2 = tuple(jnp.concatenate([lp[d][:, :5], auga[d]], axis=1)
                for d in range(3))
    pos2 = tuple(t[d] + sum(r[d][k] * lp2[k] for k in range(3))
                 for d in range(3))
    r2, t2, lpl = _frames_cols(*pos2)
    dist = jnp.sqrt(lpl[0] ** 2 + lpl[1] ** 2 + lpl[2] ** 2 + 1e-8)
    aam_ = aam[...]
    oh_aa = (lax.broadcasted_iota(jnp.int32, (aam_.shape[0], 21), 1)
             == aam_).astype(jnp.float32)
    feats = jnp.concatenate(
        [lpl[0] / (dist + 1e-8), lpl[1] / (dist + 1e-8),
         lpl[2] / (dist + 1e-8), _rbf_cols(dist, 0.0, 22.0, 16),
         jnp.log(dist + 1.0), ist[...], hot[...], oh_aa], axis=1)
    h = jax.nn.gelu(jnp.dot(feats, wl1p[...],
                            preferred_element_type=jnp.float32))
    local_o[...] = _ln(jnp.dot(h, wl2[...],
                               preferred_element_type=jnp.float32))
    lp_o[...] = jnp.concatenate(lpl, axis=1)
    r2f = jnp.concatenate([r2[d][k] for d in range(3) for k in range(3)],
                          axis=1)
    zpad = jnp.zeros((X.shape[0], GEO - (3 * NA + 9 + 1)), jnp.float32)
    geo_o[...] = jnp.concatenate(list(pos2) + [r2f, chain[...], zpad], axis=1)


def _run_prologue(X, Y, Z, ist, hot, aam, chain, w_augT, wl1p, wl2):
    grid = (N // BLK_P,)
    row = lambda i: (i, 0)
    full = lambda i: (0, 0)
    return pl.pallas_call(
        _pro_body,
        grid=grid,
        in_specs=[
            pl.BlockSpec((BLK_P, A), row), pl.BlockSpec((BLK_P, A), row),
            pl.BlockSpec((BLK_P, A), row), pl.BlockSpec((BLK_P, 1), row),
            pl.BlockSpec((BLK_P, 1), row), pl.BlockSpec((BLK_P, 1), row),
            pl.BlockSpec((BLK_P, 1), row),
            pl.BlockSpec((A, AUG), full), pl.BlockSpec((FEAT, 4 * D), full),
            pl.BlockSpec((4 * D, D), full),
        ],
        out_specs=[
            pl.BlockSpec((BLK_P, D), row), pl.BlockSpec((BLK_P, 3 * NA), row),
            pl.BlockSpec((BLK_P, GEO), row),
        ],
        out_shape=[
            jax.ShapeDtypeStruct((N, D), jnp.float32),
            jax.ShapeDtypeStruct((N, 3 * NA), jnp.float32),
            jax.ShapeDtypeStruct((N, GEO), jnp.float32),
        ],
        interpret=_INTERP,
    )(X, Y, Z, ist, hot, aam, chain, w_augT, wl1p, wl2)


BLK_K = 256


def _knn_body(cxc, cyc, czc, batc, istc, cxr, cyr, czr, batr, istr,
              nb_o, nbs_o):
    cx, cy, cz = cxc[...], cyc[...], czc[...]
    sqc = cx * cx + cy * cy + cz * cz
    rx, ry, rz = cxr[...], cyr[...], czr[...]
    sqr = rx * rx + ry * ry + rz * rz
    d2 = sqc + sqr - 2.0 * (cx * rx + cy * ry + cz * rz)
    d2 = jnp.maximum(d2, 0.0)
    keep = (batc[...] == batr[...]) & ~((istc[...] == 1) & (istr[...] == 0))
    d = jnp.where(keep, d2, BIG)
    iota = lax.broadcasted_iota(jnp.int32, d.shape, 1)
    cols = []
    for _ in range(K):
        m = jnp.min(d, axis=1, keepdims=True)
        amin = jnp.min(jnp.where(d == m, iota, N), axis=1, keepdims=True)
        cols.append(jnp.where(m < BIG / 2, amin, -1))
        d = jnp.where(iota == amin, BIG, d)
    nb = jnp.concatenate(cols, axis=1)
    nb_o[...] = nb
    nbs_o[...] = jnp.maximum(nb, 0)


def _run_knn(cxc, cyc, czc, batc, istc, cxr, cyr, czr, batr, istr):
    grid = (N // BLK_K,)
    col = lambda i: (i, 0)
    full = lambda i: (0, 0)
    return pl.pallas_call(
        _knn_body,
        grid=grid,
        in_specs=[pl.BlockSpec((BLK_K, 1), col)] * 5
        + [pl.BlockSpec((1, N), full)] * 5,
        out_specs=[pl.BlockSpec((BLK_K, K), col)] * 2,
        out_shape=[jax.ShapeDtypeStruct((N, K), jnp.int32)] * 2,
        interpret=_INTERP,
    )(cxc, cyc, czc, batc, istc, cxr, cyr, czr, batr, istr)


SC_CH = 128


def _sc_gather(table, idx):
    V, Dt = table.shape
    B = idx.shape[0]
    NW = 32
    per_w = B // NW
    chunks = per_w // SC_CH
    mesh = plsc.VectorSubcoreMesh(core_axis_name="c", subcore_axis_name="s")

    @functools.partial(
        pl.kernel, mesh=mesh,
        out_type=jax.ShapeDtypeStruct((B, Dt), jnp.float32),
        scratch_types=[
            pltpu.VMEM((SC_CH,), jnp.int32),
            pltpu.VMEM((SC_CH, Dt), jnp.float32),
            pltpu.SemaphoreType.DMA,
        ],
    )
    def k(table_hbm, idx_hbm, out_hbm, idx_v, rows_v, sem):
        wid = lax.axis_index("s") * 2 + lax.axis_index("c")
        base = wid * per_w

        def body(c, carry):
            off = base + c * SC_CH
            pltpu.sync_copy(idx_hbm.at[pl.ds(off, SC_CH)], idx_v)
            pltpu.async_copy(table_hbm.at[idx_v], rows_v, sem).wait()
            pltpu.sync_copy(rows_v, out_hbm.at[pl.ds(off, SC_CH)])
            return carry

        lax.fori_loop(0, chunks, body, 0)

    return k(table, idx)


PNODE = 128
PBLK = PNODE * K
GSLIM = 3 * NA + 9 + 1
F360 = 65 + 16 * NA + 3 * NA + 9 + 3 * NA


def _pair_body(geoT, geoT_nb, nbfT, wcatT, pm1T, pb1T, pm2T, pb2T, wbT,
               biasT_o):
    pid = pl.program_id(0)
    lane = lax.broadcasted_iota(jnp.int32, (PNODE, PBLK), 1)
    rowi = lax.broadcasted_iota(jnp.int32, (PNODE, PBLK), 0)
    E = (rowi == lane // K).astype(jnp.float32)
    gs = jnp.dot(geoT[...], E, preferred_element_type=jnp.float32)
    gn = geoT_nb[...]
    m_idx = nbfT[...]
    n_idx = (pid * PNODE
             + lax.broadcasted_iota(jnp.int32, (1, PBLK), 1) // K)
    rel = jnp.clip(jnp.maximum(m_idx, 0) - n_idx, -32, 32) + 32
    same = (gn[48:49, :] == gs[48:49, :]).astype(jnp.float32)
    oh = (lax.broadcasted_iota(jnp.int32, (65, PBLK), 0)
          == rel).astype(jnp.float32) * same
    sg = [gs[d * NA:(d + 1) * NA, :] for d in range(3)]
    ng = [gn[d * NA:(d + 1) * NA, :] for d in range(3)]
    sR = [[gs[3 * NA + d * 3 + e:3 * NA + d * 3 + e + 1, :] for e in range(3)]
          for d in range(3)]
    nR = [[gn[3 * NA + d * 3 + f:3 * NA + d * 3 + f + 1, :] for f in range(3)]
          for d in range(3)]
    diff = [sg[d] - ng[d][4:5, :] for d in range(3)]
    dd = jnp.sqrt(diff[0] ** 2 + diff[1] ** 2 + diff[2] ** 2 + 1e-8)
    c = np.linspace(0.0, 22.0, 16)
    s2 = 2 * (22.0 / 16) ** 2
    rbf = jnp.concatenate(
        [jnp.exp(-((dd - float(cj)) ** 2) / s2) for cj in c], axis=0)
    dirs = jnp.concatenate(
        [sum(diff[d] * sR[d][e] for d in range(3)) / (dd + 1e-8)
         for e in range(3)], axis=0)
    rrel = jnp.concatenate(
        [sum(sR[d][e] * nR[d][f] for d in range(3))
         for e in range(3) for f in range(3)], axis=0)
    st = [sg[d][1:2, :] for d in range(3)]
    pv = jnp.concatenate(
        [sum((ng[d] - st[d]) * sR[d][e] for d in range(3)) for e in range(3)],
        axis=0)
    f360 = jnp.concatenate([oh, rbf, dirs, rrel, pv], axis=0)
    pair = jnp.dot(wcatT[...], f360, preferred_element_type=jnp.float32)
    mu = jnp.mean(pair, axis=0, keepdims=True)
    va = jnp.var(pair, axis=0, keepdims=True)
    pair = (pair - mu) * lax.rsqrt(va + 1e-5)
    h = jax.nn.gelu(jnp.dot(pm1T[...], pair,
                            preferred_element_type=jnp.float32) + pb1T[...])
    pair = jnp.dot(pm2T[...], h, preferred_element_type=jnp.float32) + pb2T[...]
    bias = jnp.dot(wbT[...], pair, preferred_element_type=jnp.float32)
    biasT_o[...] = jnp.where(m_idx >= 0, bias, -1e9)


def _run_pair(geoT, geoT_nb, nbfT, bp):
    grid = (NK // PBLK,)
    full = lambda i: (0, 0)
    colb = lambda i: (0, i)
    return pl.pallas_call(
        _pair_body,
        grid=grid,
        in_specs=[
            pl.BlockSpec((GSLIM, PNODE), colb),
            pl.BlockSpec((GSLIM, PBLK), colb),
            pl.BlockSpec((1, PBLK), colb),
            pl.BlockSpec((P, F360), full),
            pl.BlockSpec((2 * P, P), full), pl.BlockSpec((2 * P, 1), full),
            pl.BlockSpec((P, 2 * P), full), pl.BlockSpec((P, 1), full),
            pl.BlockSpec((H, P), full),
        ],
        out_specs=pl.BlockSpec((H, PBLK), colb),
        out_shape=jax.ShapeDtypeStruct((H, NK), jnp.float32),
        interpret=_INTERP,
    )(geoT, geoT_nb, nbfT, bp['wcatT'], bp['pm1T'], bp['pb1T'], bp['pm2T'],
      bp['pb2T'], bp['wbT'])


BLK_A = 64


def _attn_body(final, local, incr, kvn, bias, lpf, wq, wo, wp1p, wp2, wu,
               wg, wout, smat, local_o, incr_o):
    loc = local[...]
    q = jnp.dot(loc, wq[...], preferred_element_type=jnp.float32)
    kvn_ = kvn[...]
    kn3 = kvn_[:, :D].reshape(BLK_A, K, D)
    vn3 = kvn_[:, D:].reshape(BLK_A, K, D)
    prod = (q[:, None, :] * kn3).reshape(BLK_A * K, D)
    sm = smat[...]
    logits = (jnp.dot(prod, sm, preferred_element_type=jnp.float32)
              .reshape(BLK_A, K, H) / np.sqrt(DH)
              + bias[...].reshape(BLK_A, K, H))
    mx = jnp.max(logits, axis=1, keepdims=True)
    e = jnp.exp(logits - mx)
    a = e / jnp.sum(e, axis=1, keepdims=True)
    arep = jnp.dot(a.reshape(BLK_A * K, H), sm.T,
                   preferred_element_type=jnp.float32).reshape(BLK_A, K, D)
    o = jnp.sum(arep * vn3, axis=1)
    up = jnp.dot(o, wo[...], preferred_element_type=jnp.float32)
    inc = incr[...] + up
    loc = _ln(loc + up)
    l2 = loc + jnp.dot(
        jax.nn.gelu(jnp.dot(lpf[...], wp1p[...],
                            preferred_element_type=jnp.float32)),
        wp2[...], preferred_element_type=jnp.float32)
    lu = jnp.dot(l2, wu[...], preferred_element_type=jnp.float32)
    lg = jax.nn.gelu(jnp.dot(l2, wg[...], preferred_element_type=jnp.float32))
    up2 = jnp.dot(lg * lu, wout[...], preferred_element_type=jnp.float32)
    inc = inc + up2
    loc = _ln(loc + up2)
    if final:
        loc = loc + _ln(inc)
    local_o[...] = loc
    incr_o[...] = inc


def _run_attn(local, incr, kvn, bias, lpf, bp, final):
    grid = (N // BLK_A,)
    row = lambda i: (i, 0)
    full = lambda i: (0, 0)
    return pl.pallas_call(
        functools.partial(_attn_body, final),
        grid=grid,
        in_specs=[
            pl.BlockSpec((BLK_A, D), row), pl.BlockSpec((BLK_A, D), row),
            pl.BlockSpec((BLK_A * K, 2 * D), row),
            pl.BlockSpec((BLK_A * K, H), row),
            pl.BlockSpec((BLK_A, 3 * NA), row),
            pl.BlockSpec((D, D), full), pl.BlockSpec((D, D), full),
            pl.BlockSpec((3 * NA, 2 * D), full), pl.BlockSpec((2 * D, D), full),
            pl.BlockSpec((D, 2 * D), full), pl.BlockSpec((D, 2 * D), full),
            pl.BlockSpec((2 * D, D), full), pl.BlockSpec((D, H), full),
        ],
        out_specs=[pl.BlockSpec((BLK_A, D), row)] * 2,
        out_shape=[jax.ShapeDtypeStruct((N, D), jnp.float32)] * 2,
        interpret=_INTERP,
    )(local, incr, kvn, bias, lpf, bp['wq'], bp['wo'], bp['wp1p'],
      bp['wp2'], bp['wu'], bp['wg'], bp['wout'], jnp.asarray(_S))


BLK_M = 512


def _kv_body(local, wkv, kv_o):
    kv_o[...] = jnp.dot(local[...], wkv[...],
                        preferred_element_type=jnp.float32)


def _run_kv(local, wkv):
    return pl.pallas_call(
        _kv_body,
        grid=(N // BLK_M,),
        in_specs=[pl.BlockSpec((BLK_M, D), lambda i: (i, 0)),
                  pl.BlockSpec((D, 2 * D), lambda i: (0, 0))],
        out_specs=pl.BlockSpec((BLK_M, 2 * D), lambda i: (i, 0)),
        out_shape=jax.ShapeDtypeStruct((N, 2 * D), jnp.float32),
        interpret=_INTERP,
    )(local, wkv)


def kernel(pos, residue_index, chain_index, batch_index, mask, is_target,
           hotspots, aa_gt, params):
    del residue_index, mask
    pp = _prep_params(params)
    X = pos[:, :, 0]
    Y = pos[:, :, 1]
    Z = pos[:, :, 2]
    is_t = is_target.astype(jnp.int32)[:, None]
    aam = jnp.where(is_target, aa_gt, 20).astype(jnp.int32)[:, None]
    chain_f = chain_index.astype(jnp.float32)[:, None]
    local, lp_flat, geo = _run_prologue(
        X, Y, Z, is_t.astype(jnp.float32),
        hotspots.astype(jnp.float32)[:, None], aam, chain_f,
        pp['w_augT'], pp['wl1p'], pp['wl2'])
    cx = geo[:, 4:5]
    cy = geo[:, NA + 4:NA + 5]
    cz = geo[:, 2 * NA + 4:2 * NA + 5]
    bat = batch_index.astype(jnp.int32)[:, None]
    nb, nbs = _run_knn(cx, cy, cz, bat, is_t,
                       cx.reshape(1, N), cy.reshape(1, N), cz.reshape(1, N),
                       bat.reshape(1, N), is_t.reshape(1, N))
    nbs_flat = nbs.reshape(NK)
    nbfT = nb.reshape(1, NK)
    geo_nb = _sc_gather(geo, nbs_flat)
    geoT = geo[:, :GSLIM].T
    geoT_nb = geo_nb[:, :GSLIM].T
    incremental = local
    for i, bp in enumerate(pp['blocks']):
        biasT = _run_pair(geoT, geoT_nb, nbfT, bp)
        bias = biasT.T
        kv = _run_kv(local, bp['wkv'])
        kvn = _sc_gather(kv, nbs_flat)
        local, incremental = _run_attn(
            local, incremental, kvn, bias, lp_flat, bp,
            final=(i == DEPTH - 1))
    return local


# speedup vs baseline: 11.2190x; 1.0348x over previous
import functools
import numpy as np
import jax
import jax.numpy as jnp
from jax import lax
from jax.experimental import pallas as pl
from jax.experimental.pallas import tpu as pltpu
from jax.experimental.pallas import tpu_sc as plsc

N = 4096; A = 14; D = 256; P = 64; K = 32; AUG = 8; H = 8; DH = 32; DEPTH = 2
NA = 5 + AUG
FEAT = NA * 3 + NA * 16 + NA + 2 + 21
BIG = 1e9
GEO = 128
NK = N * K

_INTERP = False


def _ln(x):
    m = jnp.mean(x, -1, keepdims=True)
    v = jnp.var(x, -1, keepdims=True)
    return (x - m) * lax.rsqrt(v + 1e-5)


def _frames_cols(X, Y, Z):
    nx, ny, nz = X[:, 0:1], Y[:, 0:1], Z[:, 0:1]
    cax, cay, caz = X[:, 1:2], Y[:, 1:2], Z[:, 1:2]
    cx, cy, cz = X[:, 2:3], Y[:, 2:3], Z[:, 2:3]
    e1x, e1y, e1z = cx - cax, cy - cay, cz - caz
    n1 = jnp.sqrt(e1x * e1x + e1y * e1y + e1z * e1z)
    e1x, e1y, e1z = e1x / (n1 + 1e-8), e1y / (n1 + 1e-8), e1z / (n1 + 1e-8)
    ux, uy, uz = nx - cax, ny - cay, nz - caz
    dot = ux * e1x + uy * e1y + uz * e1z
    ux, uy, uz = ux - dot * e1x, uy - dot * e1y, uz - dot * e1z
    n2 = jnp.sqrt(ux * ux + uy * uy + uz * uz)
    e2x, e2y, e2z = ux / (n2 + 1e-8), uy / (n2 + 1e-8), uz / (n2 + 1e-8)
    e3x = e1y * e2z - e1z * e2y
    e3y = e1z * e2x - e1x * e2z
    e3z = e1x * e2y - e1y * e2x
    r = ((e1x, e2x, e3x), (e1y, e2y, e3y), (e1z, e2z, e3z))
    t = (cax, cay, caz)
    dX, dY, dZ = X - cax, Y - cay, Z - caz
    lp = tuple(dX * r[0][k] + dY * r[1][k] + dZ * r[2][k] for k in range(3))
    return r, t, lp


def _rbf_cols(dd, dmin, dmax, nb):
    c = np.linspace(dmin, dmax, nb)
    s = (dmax - dmin) / nb
    return jnp.concatenate(
        [jnp.exp(-((dd - float(cj)) ** 2) / (2 * s * s)) for cj in c], axis=1)


def _perm_feats():
    p = []
    for k in range(3):
        for a in range(NA):
            p.append(a * 3 + k)
    for j in range(16):
        for a in range(NA):
            p.append(3 * NA + a * 16 + j)
    p.extend(range(3 * NA + 16 * NA, FEAT))
    return np.array(p)


def _perm_lp():
    return np.array([a * 3 + k for k in range(3) for a in range(NA)])


def _perm_cm(na, nc):
    return np.array([a * nc + j for j in range(nc) for a in range(na)])


_S = np.zeros((D, H), np.float32)
for _h in range(H):
    _S[_h * DH:(_h + 1) * DH, _h] = 1.0


def _prep_params(params):
    pf = _perm_feats()
    plp = _perm_lp()
    pcm = _perm_cm(NA, 16)
    out = {
        'w_augT': params['w_aug'].T,
        'wl1p': params['wl1'][pf],
        'wl2': params['wl2'],
        'blocks': []
    }
    for bp in params['blocks']:
        wcat = jnp.concatenate([
            bp['w_rp'], bp['w_d'][pcm], bp['w_dir'][plp], bp['w_rot'],
            bp['w_pv'][plp]], axis=0)
        out['blocks'].append({
            'wcatT': wcat.T, 'pm1T': bp['pm1'].T, 'pb1T': bp['pb1'][:, None],
            'pm2T': bp['pm2'].T, 'pb2T': bp['pb2'][:, None],
            'wbT': bp['wb'].T,
            'wq': bp['wq'], 'wkv': jnp.concatenate([bp['wk'], bp['wv']], 1),
            'wo': bp['wo'], 'wp1p': bp['wp1'][plp], 'wp2': bp['wp2'],
            'wu': bp['wu'], 'wg': bp['wg'], 'wout': bp['wout']})
    return out


BLK_P = 512


def _pro_body(X, Y, Z, ist, hot, aam, chain, w_augT, wl1p, wl2,
              local_o, lp_o, geo_o):
    X, Y, Z = X[...], Y[...], Z[...]
    r, t, lp = _frames_cols(X, Y, Z)
    w_augT_ = w_augT[...]
    aug = tuple(jnp.dot(lp[d], w_augT_, preferred_element_type=jnp.float32)
                for d in range(3))
    nrm = jnp.mean(jnp.sqrt(aug[0] ** 2 + aug[1] ** 2 + aug[2] ** 2),
                   axis=-1, keepdims=True)
    auga = tuple(aug[d] / (nrm + 1e-8) for d in range(3))
    lp2 = tuple(jnp.concatenate([lp[d][:, :5], auga[d]], axis=1)
                for d in range(3))
    pos2 = tuple(t[d] + sum(r[d][k] * lp2[k] for k in range(3))
                 for d in range(3))
    r2, t2, lpl = _frames_cols(*pos2)
    dist = jnp.sqrt(lpl[0] ** 2 + lpl[1] ** 2 + lpl[2] ** 2 + 1e-8)
    aam_ = aam[...]
    oh_aa = (lax.broadcasted_iota(jnp.int32, (aam_.shape[0], 21), 1)
             == aam_).astype(jnp.float32)
    feats = jnp.concatenate(
        [lpl[0] / (dist + 1e-8), lpl[1] / (dist + 1e-8),
         lpl[2] / (dist + 1e-8), _rbf_cols(dist, 0.0, 22.0, 16),
         jnp.log(dist + 1.0), ist[...], hot[...], oh_aa], axis=1)
    h = jax.nn.gelu(jnp.dot(feats, wl1p[...],
                            preferred_element_type=jnp.float32))
    local_o[...] = _ln(jnp.dot(h, wl2[...],
                               preferred_element_type=jnp.float32))
    lp_o[...] = jnp.concatenate(lpl, axis=1)
    r2f = jnp.concatenate([r2[d][k] for d in range(3) for k in range(3)],
                          axis=1)
    zpad = jnp.zeros((X.shape[0], GEO - (3 * NA + 9 + 1)), jnp.float32)
    geo_o[...] = jnp.concatenate(list(pos2) + [r2f, chain[...], zpad], axis=1)


def _run_prologue(X, Y, Z, ist, hot, aam, chain, w_augT, wl1p, wl2):
    grid = (N // BLK_P,)
    row = lambda i: (i, 0)
    full = lambda i: (0, 0)
    return pl.pallas_call(
        _pro_body,
        grid=grid,
        in_specs=[
            pl.BlockSpec((BLK_P, A), row), pl.BlockSpec((BLK_P, A), row),
            pl.BlockSpec((BLK_P, A), row), pl.BlockSpec((BLK_P, 1), row),
            pl.BlockSpec((BLK_P, 1), row), pl.BlockSpec((BLK_P, 1), row),
            pl.BlockSpec((BLK_P, 1), row),
            pl.BlockSpec((A, AUG), full), pl.BlockSpec((FEAT, 4 * D), full),
            pl.BlockSpec((4 * D, D), full),
        ],
        out_specs=[
            pl.BlockSpec((BLK_P, D), row), pl.BlockSpec((BLK_P, 3 * NA), row),
            pl.BlockSpec((BLK_P, GEO), row),
        ],
        out_shape=[
            jax.ShapeDtypeStruct((N, D), jnp.float32),
            jax.ShapeDtypeStruct((N, 3 * NA), jnp.float32),
            jax.ShapeDtypeStruct((N, GEO), jnp.float32),
        ],
        interpret=_INTERP,
    )(X, Y, Z, ist, hot, aam, chain, w_augT, wl1p, wl2)


BLK_K = 256


def _knn_body(cxc, cyc, czc, batc, istc, cxr, cyr, czr, batr, istr,
              nb_o, nbs_o):
    cx, cy, cz = cxc[...], cyc[...], czc[...]
    sqc = cx * cx + cy * cy + cz * cz
    rx, ry, rz = cxr[...], cyr[...], czr[...]
    sqr = rx * rx + ry * ry + rz * rz
    d2 = sqc + sqr - 2.0 * (cx * rx + cy * ry + cz * rz)
    d2 = jnp.maximum(d2, 0.0)
    keep = (batc[...] == batr[...]) & ~((istc[...] == 1) & (istr[...] == 0))
    d = jnp.where(keep, d2, BIG)
    iota = lax.broadcasted_iota(jnp.int32, d.shape, 1)
    cols = []
    for _ in range(K):
        m = jnp.min(d, axis=1, keepdims=True)
        amin = jnp.min(jnp.where(d == m, iota, N), axis=1, keepdims=True)
        cols.append(jnp.where(m < BIG / 2, amin, -1))
        d = jnp.where(iota == amin, BIG, d)
    nb = jnp.concatenate(cols, axis=1)
    nb_o[...] = nb
    nbs_o[...] = jnp.maximum(nb, 0)


def _run_knn(cxc, cyc, czc, batc, istc, cxr, cyr, czr, batr, istr):
    grid = (N // BLK_K,)
    col = lambda i: (i, 0)
    full = lambda i: (0, 0)
    return pl.pallas_call(
        _knn_body,
        grid=grid,
        in_specs=[pl.BlockSpec((BLK_K, 1), col)] * 5
        + [pl.BlockSpec((1, N), full)] * 5,
        out_specs=[pl.BlockSpec((BLK_K, K), col)] * 2,
        out_shape=[jax.ShapeDtypeStruct((N, K), jnp.int32)] * 2,
        interpret=_INTERP,
    )(cxc, cyc, czc, batc, istc, cxr, cyr, czr, batr, istr)


def _sc_gather(table, idx):
    V, Dt = table.shape
    B = idx.shape[0]
    NW = 32
    per_w = B // NW
    SC_CH = min(128, 32768 // Dt)
    chunks = per_w // SC_CH
    mesh = plsc.VectorSubcoreMesh(core_axis_name="c", subcore_axis_name="s")

    @functools.partial(
        pl.kernel, mesh=mesh,
        out_type=jax.ShapeDtypeStruct((B, Dt), jnp.float32),
        scratch_types=[
            pltpu.VMEM((SC_CH,), jnp.int32),
            pltpu.VMEM((SC_CH,), jnp.int32),
            pltpu.VMEM((SC_CH, Dt), jnp.float32),
            pltpu.VMEM((SC_CH, Dt), jnp.float32),
            pltpu.SemaphoreType.DMA,
            pltpu.SemaphoreType.DMA,
        ],
    )
    def k(table_hbm, idx_hbm, out_hbm, idx0, idx1, rows0, rows1, sem0, sem1):
        wid = lax.axis_index("s") * 2 + lax.axis_index("c")
        base = wid * per_w
        idx_b = (idx0, idx1)
        rows_b = (rows0, rows1)
        sem_b = (sem0, sem1)

        pltpu.sync_copy(idx_hbm.at[pl.ds(base, SC_CH)], idx0)
        pltpu.async_copy(table_hbm.at[idx0], rows0, sem0)

        def step(c, b, nb_):
            @pl.when(c + 1 < chunks)
            def _():
                off_n = base + (c + 1) * SC_CH
                pltpu.sync_copy(idx_hbm.at[pl.ds(off_n, SC_CH)], idx_b[nb_])
                pltpu.async_copy(table_hbm.at[idx_b[nb_]], rows_b[nb_],
                                 sem_b[nb_])
            pltpu.make_async_copy(table_hbm.at[idx_b[b]], rows_b[b],
                                  sem_b[b]).wait()
            pltpu.sync_copy(rows_b[b], out_hbm.at[pl.ds(base + c * SC_CH,
                                                        SC_CH)])

        def body(g, carry):
            step(2 * g, 0, 1)
            step(2 * g + 1, 1, 0)
            return carry

        lax.fori_loop(0, chunks // 2, body, 0)

    return k(table, idx)


PNODE = 128
PBLK = PNODE * K
GSLIM = 3 * NA + 9 + 1
F360 = 65 + 16 * NA + 3 * NA + 9 + 3 * NA


def _pair_body(geoT, geoT_nb, nbfT, wcatT, pm1T, pb1T, pm2T, pb2T, wbT,
               biasT_o):
    pid = pl.program_id(0)
    lane = lax.broadcasted_iota(jnp.int32, (PNODE, PBLK), 1)
    rowi = lax.broadcasted_iota(jnp.int32, (PNODE, PBLK), 0)
    E = (rowi == lane // K).astype(jnp.float32)
    gs = jnp.dot(geoT[...], E, preferred_element_type=jnp.float32)
    gn = geoT_nb[...]
    m_idx = nbfT[...]
    n_idx = (pid * PNODE
             + lax.broadcasted_iota(jnp.int32, (1, PBLK), 1) // K)
    rel = jnp.clip(jnp.maximum(m_idx, 0) - n_idx, -32, 32) + 32
    same = (gn[48:49, :] == gs[48:49, :]).astype(jnp.float32)
    oh = (lax.broadcasted_iota(jnp.int32, (65, PBLK), 0)
          == rel).astype(jnp.float32) * same
    sg = [gs[d * NA:(d + 1) * NA, :] for d in range(3)]
    ng = [gn[d * NA:(d + 1) * NA, :] for d in range(3)]
    sR = [[gs[3 * NA + d * 3 + e:3 * NA + d * 3 + e + 1, :] for e in range(3)]
          for d in range(3)]
    nR = [[gn[3 * NA + d * 3 + f:3 * NA + d * 3 + f + 1, :] for f in range(3)]
          for d in range(3)]
    diff = [sg[d] - ng[d][4:5, :] for d in range(3)]
    dd = jnp.sqrt(diff[0] ** 2 + diff[1] ** 2 + diff[2] ** 2 + 1e-8)
    c = np.linspace(0.0, 22.0, 16)
    s2 = 2 * (22.0 / 16) ** 2
    rbf = jnp.concatenate(
        [jnp.exp(-((dd - float(cj)) ** 2) / s2) for cj in c], axis=0)
    dirs = jnp.concatenate(
        [sum(diff[d] * sR[d][e] for d in range(3)) / (dd + 1e-8)
         for e in range(3)], axis=0)
    rrel = jnp.concatenate(
        [sum(sR[d][e] * nR[d][f] for d in range(3))
         for e in range(3) for f in range(3)], axis=0)
    st = [sg[d][1:2, :] for d in range(3)]
    pv = jnp.concatenate(
        [sum((ng[d] - st[d]) * sR[d][e] for d in range(3)) for e in range(3)],
        axis=0)
    f360 = jnp.concatenate([oh, rbf, dirs, rrel, pv], axis=0)
    pair = jnp.dot(wcatT[...], f360, preferred_element_type=jnp.float32)
    mu = jnp.mean(pair, axis=0, keepdims=True)
    va = jnp.var(pair, axis=0, keepdims=True)
    pair = (pair - mu) * lax.rsqrt(va + 1e-5)
    h = jax.nn.gelu(jnp.dot(pm1T[...], pair,
                            preferred_element_type=jnp.float32) + pb1T[...])
    pair = jnp.dot(pm2T[...], h, preferred_element_type=jnp.float32) + pb2T[...]
    bias = jnp.dot(wbT[...], pair, preferred_element_type=jnp.float32)
    biasT_o[...] = jnp.where(m_idx >= 0, bias, -1e9)


def _run_pair(geoT, geoT_nb, nbfT, bp):
    grid = (NK // PBLK,)
    full = lambda i: (0, 0)
    colb = lambda i: (0, i)
    return pl.pallas_call(
        _pair_body,
        grid=grid,
        in_specs=[
            pl.BlockSpec((GSLIM, PNODE), colb),
            pl.BlockSpec((GSLIM, PBLK), colb),
            pl.BlockSpec((1, PBLK), colb),
            pl.BlockSpec((P, F360), full),
            pl.BlockSpec((2 * P, P), full), pl.BlockSpec((2 * P, 1), full),
            pl.BlockSpec((P, 2 * P), full), pl.BlockSpec((P, 1), full),
            pl.BlockSpec((H, P), full),
        ],
        out_specs=pl.BlockSpec((H, PBLK), colb),
        out_shape=jax.ShapeDtypeStruct((H, NK), jnp.float32),
        interpret=_INTERP,
    )(geoT, geoT_nb, nbfT, bp['wcatT'], bp['pm1T'], bp['pb1T'], bp['pm2T'],
      bp['pb2T'], bp['wbT'])


BLK_A = 64


def _attn_body(final, local, incr, kvn, bias, lpf, wq, wo, wp1p, wp2, wu,
               wg, wout, smat, local_o, incr_o):
    loc = local[...]
    q = jnp.dot(loc, wq[...], preferred_element_type=jnp.float32)
    kvn_ = kvn[...]
    kn3 = kvn_[:, :D].reshape(BLK_A, K, D)
    vn3 = kvn_[:, D:].reshape(BLK_A, K, D)
    prod = (q[:, None, :] * kn3).reshape(BLK_A * K, D)
    sm = smat[...]
    logits = (jnp.dot(prod, sm, preferred_element_type=jnp.float32)
              .reshape(BLK_A, K, H) / np.sqrt(DH)
              + bias[...].reshape(BLK_A, K, H))
    mx = jnp.max(logits, axis=1, keepdims=True)
    e = jnp.exp(logits - mx)
    a = e / jnp.sum(e, axis=1, keepdims=True)
    arep = jnp.dot(a.reshape(BLK_A * K, H), sm.T,
                   preferred_element_type=jnp.float32).reshape(BLK_A, K, D)
    o = jnp.sum(arep * vn3, axis=1)
    up = jnp.dot(o, wo[...], preferred_element_type=jnp.float32)
    inc = incr[...] + up
    loc = _ln(loc + up)
    l2 = loc + jnp.dot(
        jax.nn.gelu(jnp.dot(lpf[...], wp1p[...],
                            preferred_element_type=jnp.float32)),
        wp2[...], preferred_element_type=jnp.float32)
    lu = jnp.dot(l2, wu[...], preferred_element_type=jnp.float32)
    lg = jax.nn.gelu(jnp.dot(l2, wg[...], preferred_element_type=jnp.float32))
    up2 = jnp.dot(lg * lu, wout[...], preferred_element_type=jnp.float32)
    inc = inc + up2
    loc = _ln(loc + up2)
    if final:
        loc = loc + _ln(inc)
    local_o[...] = loc
    incr_o[...] = inc


def _run_attn(local, incr, kvn, bias, lpf, bp, final):
    grid = (N // BLK_A,)
    row = lambda i: (i, 0)
    full = lambda i: (0, 0)
    return pl.pallas_call(
        functools.partial(_attn_body, final),
        grid=grid,
        in_specs=[
            pl.BlockSpec((BLK_A, D), row), pl.BlockSpec((BLK_A, D), row),
            pl.BlockSpec((BLK_A * K, 2 * D), row),
            pl.BlockSpec((BLK_A * K, H), row),
            pl.BlockSpec((BLK_A, 3 * NA), row),
            pl.BlockSpec((D, D), full), pl.BlockSpec((D, D), full),
            pl.BlockSpec((3 * NA, 2 * D), full), pl.BlockSpec((2 * D, D), full),
            pl.BlockSpec((D, 2 * D), full), pl.BlockSpec((D, 2 * D), full),
            pl.BlockSpec((2 * D, D), full), pl.BlockSpec((D, H), full),
        ],
        out_specs=[pl.BlockSpec((BLK_A, D), row)] * 2,
        out_shape=[jax.ShapeDtypeStruct((N, D), jnp.float32)] * 2,
        interpret=_INTERP,
    )(local, incr, kvn, bias, lpf, bp['wq'], bp['wo'], bp['wp1p'],
      bp['wp2'], bp['wu'], bp['wg'], bp['wout'], jnp.asarray(_S))


BLK_M = 512


def _kv_body(local, wkv, kv_o):
    kv_o[...] = jnp.dot(local[...], wkv[...],
                        preferred_element_type=jnp.float32)


def _run_kv(local, wkv):
    return pl.pallas_call(
        _kv_body,
        grid=(N // BLK_M,),
        in_specs=[pl.BlockSpec((BLK_M, D), lambda i: (i, 0)),
                  pl.BlockSpec((D, 2 * D), lambda i: (0, 0))],
        out_specs=pl.BlockSpec((BLK_M, 2 * D), lambda i: (i, 0)),
        out_shape=jax.ShapeDtypeStruct((N, 2 * D), jnp.float32),
        interpret=_INTERP,
    )(local, wkv)


def kernel(pos, residue_index, chain_index, batch_index, mask, is_target,
           hotspots, aa_gt, params):
    del residue_index, mask
    pp = _prep_params(params)
    X = pos[:, :, 0]
    Y = pos[:, :, 1]
    Z = pos[:, :, 2]
    is_t = is_target.astype(jnp.int32)[:, None]
    aam = jnp.where(is_target, aa_gt, 20).astype(jnp.int32)[:, None]
    chain_f = chain_index.astype(jnp.float32)[:, None]
    local, lp_flat, geo = _run_prologue(
        X, Y, Z, is_t.astype(jnp.float32),
        hotspots.astype(jnp.float32)[:, None], aam, chain_f,
        pp['w_augT'], pp['wl1p'], pp['wl2'])
    cx = geo[:, 4:5]
    cy = geo[:, NA + 4:NA + 5]
    cz = geo[:, 2 * NA + 4:2 * NA + 5]
    bat = batch_index.astype(jnp.int32)[:, None]
    nb, nbs = _run_knn(cx, cy, cz, bat, is_t,
                       cx.reshape(1, N), cy.reshape(1, N), cz.reshape(1, N),
                       bat.reshape(1, N), is_t.reshape(1, N))
    nbs_flat = nbs.reshape(NK)
    nbfT = nb.reshape(1, NK)
    geo_nb = _sc_gather(geo, nbs_flat)
    geoT = geo[:, :GSLIM].T
    geoT_nb = geo_nb[:, :GSLIM].T
    incremental = local
    for i, bp in enumerate(pp['blocks']):
        kv = _run_kv(local, bp['wkv'])
        kvn = _sc_gather(kv, nbs_flat)
        biasT = _run_pair(geoT, geoT_nb, nbfT, bp)
        bias = biasT.T
        local, incremental = _run_attn(
            local, incremental, kvn, bias, lp_flat, bp,
            final=(i == DEPTH - 1))
    return local


# speedup vs baseline: 11.6372x; 1.0373x over previous
import functools
import numpy as np
import jax
import jax.numpy as jnp
from jax import lax
from jax.experimental import pallas as pl
from jax.experimental.pallas import tpu as pltpu
from jax.experimental.pallas import tpu_sc as plsc

N = 4096; A = 14; D = 256; P = 64; K = 32; AUG = 8; H = 8; DH = 32; DEPTH = 2
NA = 5 + AUG
FEAT = NA * 3 + NA * 16 + NA + 2 + 21
BIG = 1e9
GEO = 128
NK = N * K


def _ln(x):
    m = jnp.mean(x, -1, keepdims=True)
    v = jnp.var(x, -1, keepdims=True)
    return (x - m) * lax.rsqrt(v + 1e-5)


def _frames_cols(X, Y, Z):
    nx, ny, nz = X[:, 0:1], Y[:, 0:1], Z[:, 0:1]
    cax, cay, caz = X[:, 1:2], Y[:, 1:2], Z[:, 1:2]
    cx, cy, cz = X[:, 2:3], Y[:, 2:3], Z[:, 2:3]
    e1x, e1y, e1z = cx - cax, cy - cay, cz - caz
    n1 = jnp.sqrt(e1x * e1x + e1y * e1y + e1z * e1z)
    e1x, e1y, e1z = e1x / (n1 + 1e-8), e1y / (n1 + 1e-8), e1z / (n1 + 1e-8)
    ux, uy, uz = nx - cax, ny - cay, nz - caz
    dot = ux * e1x + uy * e1y + uz * e1z
    ux, uy, uz = ux - dot * e1x, uy - dot * e1y, uz - dot * e1z
    n2 = jnp.sqrt(ux * ux + uy * uy + uz * uz)
    e2x, e2y, e2z = ux / (n2 + 1e-8), uy / (n2 + 1e-8), uz / (n2 + 1e-8)
    e3x = e1y * e2z - e1z * e2y
    e3y = e1z * e2x - e1x * e2z
    e3z = e1x * e2y - e1y * e2x
    r = ((e1x, e2x, e3x), (e1y, e2y, e3y), (e1z, e2z, e3z))
    t = (cax, cay, caz)
    dX, dY, dZ = X - cax, Y - cay, Z - caz
    lp = tuple(dX * r[0][k] + dY * r[1][k] + dZ * r[2][k] for k in range(3))
    return r, t, lp


def _rbf_cols(dd, dmin, dmax, nb):
    c = np.linspace(dmin, dmax, nb)
    s = (dmax - dmin) / nb
    return jnp.concatenate(
        [jnp.exp(-((dd - float(cj)) ** 2) / (2 * s * s)) for cj in c], axis=1)


def _perm_feats():
    p = []
    for k in range(3):
        for a in range(NA):
            p.append(a * 3 + k)
    for j in range(16):
        for a in range(NA):
            p.append(3 * NA + a * 16 + j)
    p.extend(range(3 * NA + 16 * NA, FEAT))
    return np.array(p)


def _perm_lp():
    return np.array([a * 3 + k for k in range(3) for a in range(NA)])


def _perm_cm(na, nc):
    return np.array([a * nc + j for j in range(nc) for a in range(na)])


_S = np.zeros((D, H), np.float32)
for _h in range(H):
    _S[_h * DH:(_h + 1) * DH, _h] = 1.0


def _prep_params(params):
    pf = _perm_feats()
    plp = _perm_lp()
    pcm = _perm_cm(NA, 16)
    out = {
        'w_augT': params['w_aug'].T,
        'wl1p': params['wl1'][pf],
        'wl2': params['wl2'],
        'blocks': []
    }
    for bp in params['blocks']:
        wcat = jnp.concatenate([
            bp['w_rp'], bp['w_d'][pcm], bp['w_dir'][plp], bp['w_rot'],
            bp['w_pv'][plp]], axis=0)
        out['blocks'].append({
            'wcatT': wcat.T, 'pm1T': bp['pm1'].T, 'pb1T': bp['pb1'][:, None],
            'pm2T': bp['pm2'].T, 'pb2T': bp['pb2'][:, None],
            'wbT': bp['wb'].T,
            'wq': bp['wq'], 'wkv': jnp.concatenate([bp['wk'], bp['wv']], 1),
            'wo': bp['wo'], 'wp1p': bp['wp1'][plp], 'wp2': bp['wp2'],
            'wu': bp['wu'], 'wg': bp['wg'], 'wout': bp['wout']})
    return out


BLK_P = 512


def _pro_body(X, Y, Z, ist, hot, aam, chain, w_augT, wl1p, wl2,
              local_o, lp_o, geo_o):
    X, Y, Z = X[...], Y[...], Z[...]
    r, t, lp = _frames_cols(X, Y, Z)
    w_augT_ = w_augT[...]
    aug = tuple(jnp.dot(lp[d], w_augT_, preferred_element_type=jnp.float32)
                for d in range(3))
    nrm = jnp.mean(jnp.sqrt(aug[0] ** 2 + aug[1] ** 2 + aug[2] ** 2),
                   axis=-1, keepdims=True)
    auga = tuple(aug[d] / (nrm + 1e-8) for d in range(3))
    lp2 = tuple(jnp.concatenate([lp[d][:, :5], auga[d]], axis=1)
                for d in range(3))
    pos2 = tuple(t[d] + sum(r[d][k] * lp2[k] for k in range(3))
                 for d in range(3))
    r2, t2, lpl = _frames_cols(*pos2)
    dist = jnp.sqrt(lpl[0] ** 2 + lpl[1] ** 2 + lpl[2] ** 2 + 1e-8)
    aam_ = aam[...]
    oh_aa = (lax.broadcasted_iota(jnp.int32, (aam_.shape[0], 21), 1)
             == aam_).astype(jnp.float32)
    feats = jnp.concatenate(
        [lpl[0] / (dist + 1e-8), lpl[1] / (dist + 1e-8),
         lpl[2] / (dist + 1e-8), _rbf_cols(dist, 0.0, 22.0, 16),
         jnp.log(dist + 1.0), ist[...], hot[...], oh_aa], axis=1)
    h = jax.nn.gelu(jnp.dot(feats, wl1p[...],
                            preferred_element_type=jnp.float32))
    local_o[...] = _ln(jnp.dot(h, wl2[...],
                               preferred_element_type=jnp.float32))
    lp_o[...] = jnp.concatenate(lpl, axis=1)
    r2f = jnp.concatenate([r2[d][k] for d in range(3) for k in range(3)],
                          axis=1)
    zpad = jnp.zeros((X.shape[0], GEO - (3 * NA + 9 + 1)), jnp.float32)
    geo_o[...] = jnp.concatenate(list(pos2) + [r2f, chain[...], zpad], axis=1)


def _run_prologue(X, Y, Z, ist, hot, aam, chain, w_augT, wl1p, wl2):
    grid = (N // BLK_P,)
    row = lambda i: (i, 0)
    full = lambda i: (0, 0)
    return pl.pallas_call(
        _pro_body,
        grid=grid,
        in_specs=[
            pl.BlockSpec((BLK_P, A), row), pl.BlockSpec((BLK_P, A), row),
            pl.BlockSpec((BLK_P, A), row), pl.BlockSpec((BLK_P, 1), row),
            pl.BlockSpec((BLK_P, 1), row), pl.BlockSpec((BLK_P, 1), row),
            pl.BlockSpec((BLK_P, 1), row),
            pl.BlockSpec((A, AUG), full), pl.BlockSpec((FEAT, 4 * D), full),
            pl.BlockSpec((4 * D, D), full),
        ],
        out_specs=[
            pl.BlockSpec((BLK_P, D), row), pl.BlockSpec((BLK_P, 3 * NA), row),
            pl.BlockSpec((BLK_P, GEO), row),
        ],
        out_shape=[
            jax.ShapeDtypeStruct((N, D), jnp.float32),
            jax.ShapeDtypeStruct((N, 3 * NA), jnp.float32),
            jax.ShapeDtypeStruct((N, GEO), jnp.float32),
        ],
    )(X, Y, Z, ist, hot, aam, chain, w_augT, wl1p, wl2)


BLK_K = 256


def _knn_body(cxc, cyc, czc, batc, istc, cxr, cyr, czr, batr, istr,
              nb_o, nbs_o):
    cx, cy, cz = cxc[...], cyc[...], czc[...]
    sqc = cx * cx + cy * cy + cz * cz
    rx, ry, rz = cxr[...], cyr[...], czr[...]
    sqr = rx * rx + ry * ry + rz * rz
    d2 = sqc + sqr - 2.0 * (cx * rx + cy * ry + cz * rz)
    d2 = jnp.maximum(d2, 0.0)
    keep = (batc[...] == batr[...]) & ~((istc[...] == 1) & (istr[...] == 0))
    d = jnp.where(keep, d2, BIG)
    iota = lax.broadcasted_iota(jnp.int32, d.shape, 1)
    cols = []
    for _ in range(K):
        m = jnp.min(d, axis=1, keepdims=True)
        amin = jnp.min(jnp.where(d == m, iota, N), axis=1, keepdims=True)
        cols.append(jnp.where(m < BIG / 2, amin, -1))
        d = jnp.where(iota == amin, BIG, d)
    nb = jnp.concatenate(cols, axis=1)
    nb_o[...] = nb
    nbs_o[...] = jnp.maximum(nb, 0)


def _run_knn(cxc, cyc, czc, batc, istc, cxr, cyr, czr, batr, istr):
    grid = (N // BLK_K,)
    col = lambda i: (i, 0)
    full = lambda i: (0, 0)
    return pl.pallas_call(
        _knn_body,
        grid=grid,
        in_specs=[pl.BlockSpec((BLK_K, 1), col)] * 5
        + [pl.BlockSpec((1, N), full)] * 5,
        out_specs=[pl.BlockSpec((BLK_K, K), col)] * 2,
        out_shape=[jax.ShapeDtypeStruct((N, K), jnp.int32)] * 2,
    )(cxc, cyc, czc, batc, istc, cxr, cyr, czr, batr, istr)


def _sc_gather(table, idx):
    V, Dt = table.shape
    B = idx.shape[0]
    NW = 32
    per_w = B // NW
    SC_CH = min(128, 32768 // Dt)
    chunks = per_w // SC_CH
    mesh = plsc.VectorSubcoreMesh(core_axis_name="c", subcore_axis_name="s")

    @functools.partial(
        pl.kernel, mesh=mesh,
        out_type=jax.ShapeDtypeStruct((B, Dt), jnp.float32),
        scratch_types=[
            pltpu.VMEM((SC_CH,), jnp.int32),
            pltpu.VMEM((SC_CH,), jnp.int32),
            pltpu.VMEM((SC_CH, Dt), jnp.float32),
            pltpu.VMEM((SC_CH, Dt), jnp.float32),
            pltpu.SemaphoreType.DMA,
            pltpu.SemaphoreType.DMA,
        ],
    )
    def k(table_hbm, idx_hbm, out_hbm, idx0, idx1, rows0, rows1, sem0, sem1):
        wid = lax.axis_index("s") * 2 + lax.axis_index("c")
        base = wid * per_w
        idx_b = (idx0, idx1)
        rows_b = (rows0, rows1)
        sem_b = (sem0, sem1)

        pltpu.sync_copy(idx_hbm.at[pl.ds(base, SC_CH)], idx0)
        pltpu.async_copy(table_hbm.at[idx0], rows0, sem0)

        def step(c, b, nb_):
            @pl.when(c + 1 < chunks)
            def _():
                off_n = base + (c + 1) * SC_CH
                pltpu.sync_copy(idx_hbm.at[pl.ds(off_n, SC_CH)], idx_b[nb_])
                pltpu.async_copy(table_hbm.at[idx_b[nb_]], rows_b[nb_],
                                 sem_b[nb_])
            pltpu.make_async_copy(table_hbm.at[idx_b[b]], rows_b[b],
                                  sem_b[b]).wait()
            pltpu.sync_copy(rows_b[b], out_hbm.at[pl.ds(base + c * SC_CH,
                                                        SC_CH)])

        def body(g, carry):
            step(2 * g, 0, 1)
            step(2 * g + 1, 1, 0)
            return carry

        lax.fori_loop(0, chunks // 2, body, 0)

    return k(table, idx)


PNODE = 128
PBLK = PNODE * K
GSLIM = 3 * NA + 9 + 1
F360 = 65 + 16 * NA + 3 * NA + 9 + 3 * NA


def _pair_body(geoT, geoT_nb, nbfT, wcatT, pm1T, pb1T, pm2T, pb2T, wbT,
               biasT_o):
    pid = pl.program_id(0)
    lane = lax.broadcasted_iota(jnp.int32, (PNODE, PBLK), 1)
    rowi = lax.broadcasted_iota(jnp.int32, (PNODE, PBLK), 0)
    E = (rowi == lane // K).astype(jnp.float32)
    gs = jnp.dot(geoT[...], E, preferred_element_type=jnp.float32)
    gn = geoT_nb[...]
    m_idx = nbfT[...]
    n_idx = (pid * PNODE
             + lax.broadcasted_iota(jnp.int32, (1, PBLK), 1) // K)
    rel = jnp.clip(jnp.maximum(m_idx, 0) - n_idx, -32, 32) + 32
    same = (gn[48:49, :] == gs[48:49, :]).astype(jnp.float32)
    oh = (lax.broadcasted_iota(jnp.int32, (65, PBLK), 0)
          == rel).astype(jnp.float32) * same
    sg = [gs[d * NA:(d + 1) * NA, :] for d in range(3)]
    ng = [gn[d * NA:(d + 1) * NA, :] for d in range(3)]
    sR = [[gs[3 * NA + d * 3 + e:3 * NA + d * 3 + e + 1, :] for e in range(3)]
          for d in range(3)]
    nR = [[gn[3 * NA + d * 3 + f:3 * NA + d * 3 + f + 1, :] for f in range(3)]
          for d in range(3)]
    diff = [sg[d] - ng[d][4:5, :] for d in range(3)]
    dd = jnp.sqrt(diff[0] ** 2 + diff[1] ** 2 + diff[2] ** 2 + 1e-8)
    c = np.linspace(0.0, 22.0, 16)
    s2 = 2 * (22.0 / 16) ** 2
    rbf = jnp.concatenate(
        [jnp.exp(-((dd - float(cj)) ** 2) / s2) for cj in c], axis=0)
    dirs = jnp.concatenate(
        [sum(diff[d] * sR[d][e] for d in range(3)) / (dd + 1e-8)
         for e in range(3)], axis=0)
    rrel = jnp.concatenate(
        [sum(sR[d][e] * nR[d][f] for d in range(3))
         for e in range(3) for f in range(3)], axis=0)
    st = [sg[d][1:2, :] for d in range(3)]
    pv = jnp.concatenate(
        [sum((ng[d] - st[d]) * sR[d][e] for d in range(3)) for e in range(3)],
        axis=0)
    f360 = jnp.concatenate([oh, rbf, dirs, rrel, pv], axis=0)
    pair = jnp.dot(wcatT[...], f360, preferred_element_type=jnp.float32)
    mu = jnp.mean(pair, axis=0, keepdims=True)
    va = jnp.var(pair, axis=0, keepdims=True)
    pair = (pair - mu) * lax.rsqrt(va + 1e-5)
    h = jax.nn.gelu(jnp.dot(pm1T[...], pair,
                            preferred_element_type=jnp.float32) + pb1T[...])
    pair = jnp.dot(pm2T[...], h, preferred_element_type=jnp.float32) + pb2T[...]
    bias = jnp.dot(wbT[...], pair, preferred_element_type=jnp.float32)
    biasT_o[...] = jnp.where(m_idx >= 0, bias, -1e9)


def _run_pair(geoT, geoT_nb, nbfT, bp):
    grid = (NK // PBLK,)
    full = lambda i: (0, 0)
    colb = lambda i: (0, i)
    return pl.pallas_call(
        _pair_body,
        grid=grid,
        in_specs=[
            pl.BlockSpec((GSLIM, PNODE), colb),
            pl.BlockSpec((GSLIM, PBLK), colb),
            pl.BlockSpec((1, PBLK), colb),
            pl.BlockSpec((P, F360), full),
            pl.BlockSpec((2 * P, P), full), pl.BlockSpec((2 * P, 1), full),
            pl.BlockSpec((P, 2 * P), full), pl.BlockSpec((P, 1), full),
            pl.BlockSpec((H, P), full),
        ],
        out_specs=pl.BlockSpec((H, PBLK), colb),
        out_shape=jax.ShapeDtypeStruct((H, NK), jnp.float32),
    )(geoT, geoT_nb, nbfT, bp['wcatT'], bp['pm1T'], bp['pb1T'], bp['pm2T'],
      bp['pb2T'], bp['wbT'])


BLK_A = 128


def _attn_body(final, local, incr, kvn, bias, lpf, wq, wo, wp1p, wp2, wu,
               wg, wout, smat, local_o, incr_o):
    loc = local[...]
    q = jnp.dot(loc, wq[...], preferred_element_type=jnp.float32)
    kvn_ = kvn[...]
    kn3 = kvn_[:, :D].reshape(BLK_A, K, D)
    vn3 = kvn_[:, D:].reshape(BLK_A, K, D)
    prod = (q[:, None, :] * kn3).reshape(BLK_A * K, D)
    sm = smat[...]
    logits = (jnp.dot(prod, sm, preferred_element_type=jnp.float32)
              .reshape(BLK_A, K, H) / np.sqrt(DH)
              + bias[...].reshape(BLK_A, K, H))
    mx = jnp.max(logits, axis=1, keepdims=True)
    e = jnp.exp(logits - mx)
    a = e / jnp.sum(e, axis=1, keepdims=True)
    arep = jnp.dot(a.reshape(BLK_A * K, H), sm.T,
                   preferred_element_type=jnp.float32).reshape(BLK_A, K, D)
    o = jnp.sum(arep * vn3, axis=1)
    up = jnp.dot(o, wo[...], preferred_element_type=jnp.float32)
    inc = incr[...] + up
    loc = _ln(loc + up)
    l2 = loc + jnp.dot(
        jax.nn.gelu(jnp.dot(lpf[...], wp1p[...],
                            preferred_element_type=jnp.float32)),
        wp2[...], preferred_element_type=jnp.float32)
    lu = jnp.dot(l2, wu[...], preferred_element_type=jnp.float32)
    lg = jax.nn.gelu(jnp.dot(l2, wg[...], preferred_element_type=jnp.float32))
    up2 = jnp.dot(lg * lu, wout[...], preferred_element_type=jnp.float32)
    inc = inc + up2
    loc = _ln(loc + up2)
    if final:
        loc = loc + _ln(inc)
    local_o[...] = loc
    incr_o[...] = inc


def _run_attn(local, incr, kvn, bias, lpf, bp, final):
    grid = (N // BLK_A,)
    row = lambda i: (i, 0)
    full = lambda i: (0, 0)
    return pl.pallas_call(
        functools.partial(_attn_body, final),
        grid=grid,
        in_specs=[
            pl.BlockSpec((BLK_A, D), row), pl.BlockSpec((BLK_A, D), row),
            pl.BlockSpec((BLK_A * K, 2 * D), row),
            pl.BlockSpec((BLK_A * K, H), row),
            pl.BlockSpec((BLK_A, 3 * NA), row),
            pl.BlockSpec((D, D), full), pl.BlockSpec((D, D), full),
            pl.BlockSpec((3 * NA, 2 * D), full), pl.BlockSpec((2 * D, D), full),
            pl.BlockSpec((D, 2 * D), full), pl.BlockSpec((D, 2 * D), full),
            pl.BlockSpec((2 * D, D), full), pl.BlockSpec((D, H), full),
        ],
        out_specs=[pl.BlockSpec((BLK_A, D), row)] * 2,
        out_shape=[jax.ShapeDtypeStruct((N, D), jnp.float32)] * 2,
    )(local, incr, kvn, bias, lpf, bp['wq'], bp['wo'], bp['wp1p'],
      bp['wp2'], bp['wu'], bp['wg'], bp['wout'], jnp.asarray(_S))


BLK_M = 512


def _kv_body(local, wkv, kv_o):
    kv_o[...] = jnp.dot(local[...], wkv[...],
                        preferred_element_type=jnp.float32)


def _run_kv(local, wkv):
    return pl.pallas_call(
        _kv_body,
        grid=(N // BLK_M,),
        in_specs=[pl.BlockSpec((BLK_M, D), lambda i: (i, 0)),
                  pl.BlockSpec((D, 2 * D), lambda i: (0, 0))],
        out_specs=pl.BlockSpec((BLK_M, 2 * D), lambda i: (i, 0)),
        out_shape=jax.ShapeDtypeStruct((N, 2 * D), jnp.float32),
    )(local, wkv)


def kernel(pos, residue_index, chain_index, batch_index, mask, is_target,
           hotspots, aa_gt, params):
    del residue_index, mask
    pp = _prep_params(params)
    X = pos[:, :, 0]
    Y = pos[:, :, 1]
    Z = pos[:, :, 2]
    is_t = is_target.astype(jnp.int32)[:, None]
    aam = jnp.where(is_target, aa_gt, 20).astype(jnp.int32)[:, None]
    chain_f = chain_index.astype(jnp.float32)[:, None]
    local, lp_flat, geo = _run_prologue(
        X, Y, Z, is_t.astype(jnp.float32),
        hotspots.astype(jnp.float32)[:, None], aam, chain_f,
        pp['w_augT'], pp['wl1p'], pp['wl2'])
    cx = geo[:, 4:5]
    cy = geo[:, NA + 4:NA + 5]
    cz = geo[:, 2 * NA + 4:2 * NA + 5]
    bat = batch_index.astype(jnp.int32)[:, None]
    nb, nbs = _run_knn(cx, cy, cz, bat, is_t,
                       cx.reshape(1, N), cy.reshape(1, N), cz.reshape(1, N),
                       bat.reshape(1, N), is_t.reshape(1, N))
    nbs_flat = nbs.reshape(NK)
    nbfT = nb.reshape(1, NK)
    geo_nb = _sc_gather(geo, nbs_flat)
    geoT = geo[:, :GSLIM].T
    geoT_nb = geo_nb[:, :GSLIM].T
    incremental = local
    for i, bp in enumerate(pp['blocks']):
        kv = _run_kv(local, bp['wkv'])
        kvn = _sc_gather(kv, nbs_flat)
        biasT = _run_pair(geoT, geoT_nb, nbfT, bp)
        bias = biasT.T
        local, incremental = _run_attn(
            local, incremental, kvn, bias, lp_flat, bp,
            final=(i == DEPTH - 1))
    return local
